# baseline (speedup 1.0000x reference)
import functools

import jax
import jax.numpy as jnp
from jax import lax
from jax.experimental import pallas as pl
from jax.experimental.pallas import tpu as pltpu
from jax.experimental.pallas import tpu_sc as plsc

N_FIELDS = 26
BATCH = 16384
DIM = 36
VOCAB = 100000
VA = 49920
VTAIL = VOCAB - 2 * VA
TROW = 256
NC = 2
NS = 16
NW = NC * NS
NU = N_FIELDS * DIM
HB = BATCH // 2
LANES = 16

_mesh = plsc.VectorSubcoreMesh(core_axis_name="c", subcore_axis_name="s")


@functools.partial(
    pl.kernel,
    mesh=_mesh,
    out_type=jax.ShapeDtypeStruct((NU, BATCH), jnp.float32),
    scratch_types=[
        pltpu.VMEM((VA,), jnp.float32),
        pltpu.VMEM((VOCAB - VA - VTAIL + TROW,), jnp.float32),
        pltpu.VMEM((BATCH,), jnp.int32),
        pltpu.VMEM((HB,), jnp.float32),
        pltpu.SemaphoreType.DMA,
        pltpu.SemaphoreType.DMA,
    ],
    compiler_params=pltpu.CompilerParams(needs_layout_passes=False),
)
def _encode(xt_hbm, *rest):
    tables = rest[:N_FIELDS]
    tails = rest[N_FIELDS:2 * N_FIELDS]
    out_hbm, rowa_v, rowb_v, idx_v, och_v, sema, semb = rest[2 * N_FIELDS:]
    wid = lax.axis_index("s") * NC + lax.axis_index("c")
    lo_u = (wid * NU) // NW
    hi_u = ((wid + 1) * NU) // NW
    UNROLL = 8

    def pass_a(hoff):
        def gbody(c, _):
            base = c * (LANES * UNROLL)
            for u in range(UNROLL):
                sl = pl.ds(base + u * LANES, LANES)
                iv = idx_v[pl.ds(hoff + base + u * LANES, LANES)]
                och_v[sl] = plsc.load_gather(rowa_v, [iv], mask=iv < VA)
            return 0
        lax.fori_loop(0, HB // (LANES * UNROLL), gbody, 0)

    def pass_b(hoff):
        def gbody(c, _):
            base = c * (LANES * UNROLL)
            for u in range(UNROLL):
                sl = pl.ds(base + u * LANES, LANES)
                iv = idx_v[pl.ds(hoff + base + u * LANES, LANES)]
                m = iv >= VA
                vb = plsc.load_gather(rowb_v, [iv - VA], mask=m)
                och_v[sl] = jnp.where(m, vb, och_v[sl])
            return 0
        lax.fori_loop(0, HB // (LANES * UNROLL), gbody, 0)

    VB = VOCAB - VA - VTAIL

    def start_b(tbl, tt, d):
        pltpu.async_copy(tbl.at[d].at[pl.ds(VA, VB)], rowb_v.at[pl.ds(0, VB)],
                         semb)
        pltpu.async_copy(tt.at[pl.ds(d * TROW, TROW)],
                         rowb_v.at[pl.ds(VB, TROW)], semb)

    def wait_b(tbl, tt, d):
        pltpu.make_async_copy(tbl.at[d].at[pl.ds(VA, VB)],
                              rowb_v.at[pl.ds(0, VB)], semb).wait()
        pltpu.make_async_copy(tt.at[pl.ds(d * TROW, TROW)],
                              rowb_v.at[pl.ds(VB, TROW)], semb).wait()

    for f in range(N_FIELDS):
        tbl = tables[f]
        tt = tails[f]
        dlo = jnp.clip(lo_u - f * DIM, 0, DIM)
        dhi = jnp.clip(hi_u - f * DIM, 0, DIM)

        @pl.when(dhi > dlo)
        def _(f=f, tbl=tbl, tt=tt, dlo=dlo):
            pltpu.sync_copy(xt_hbm.at[f], idx_v)
            pltpu.async_copy(tbl.at[dlo].at[pl.ds(0, VA)], rowa_v, sema)
            start_b(tbl, tt, dlo)

        def dbody(d, _, f=f, tbl=tbl, tt=tt, dhi=dhi):
            pltpu.make_async_copy(tbl.at[d].at[pl.ds(0, VA)], rowa_v,
                                  sema).wait()
            wait_b(tbl, tt, d)

            def hbody(h, _, f=f, tbl=tbl, tt=tt, dhi=dhi, d=d):
                pass_a(h * HB)

                @pl.when((h == 1) & (d + 1 < dhi))
                def _():
                    pltpu.async_copy(tbl.at[d + 1].at[pl.ds(0, VA)], rowa_v,
                                     sema)

                pass_b(h * HB)

                @pl.when((h == 1) & (d + 1 < dhi))
                def _():
                    start_b(tbl, tt, d + 1)

                pltpu.sync_copy(och_v,
                                out_hbm.at[f * DIM + d, pl.ds(h * HB, HB)])
                return 0

            lax.fori_loop(0, 2, hbody, 0)
            return 0

        lax.fori_loop(dlo, dhi, dbody, 0)


def kernel(x, table_0, table_1, table_2, table_3, table_4, table_5, table_6,
           table_7, table_8, table_9, table_10, table_11, table_12, table_13,
           table_14, table_15, table_16, table_17, table_18, table_19,
           table_20, table_21, table_22, table_23, table_24, table_25):
    tables = (table_0, table_1, table_2, table_3, table_4, table_5, table_6,
              table_7, table_8, table_9, table_10, table_11, table_12,
              table_13, table_14, table_15, table_16, table_17, table_18,
              table_19, table_20, table_21, table_22, table_23, table_24,
              table_25)
    tails = tuple(
        jnp.pad(t.T[:, VOCAB - VTAIL:], ((0, 0), (0, TROW - VTAIL)))
        .reshape(-1) for t in tables)
    out_t = _encode(x.T, *(t.T for t in tables), *tails)
    return out_t.T.reshape(BATCH, N_FIELDS * DIM)


# speedup vs baseline: 2.4535x; 2.4535x over previous
import functools

import jax
import jax.numpy as jnp
from jax import lax
from jax.experimental import pallas as pl
from jax.experimental.pallas import tpu as pltpu
from jax.experimental.pallas import tpu_sc as plsc

N_FIELDS = 26
BATCH = 16384
DIM = 36
VOCAB = 100000
NC = 2
NS = 16
NW = NC * NS
NU = N_FIELDS * DIM
HB = BATCH // 2
LANES = 16

_mesh = plsc.VectorSubcoreMesh(core_axis_name="c", subcore_axis_name="s")


@functools.partial(
    pl.kernel,
    mesh=_mesh,
    out_type=jax.ShapeDtypeStruct((NU, BATCH), jnp.float32),
    scratch_types=[
        pltpu.VMEM((VOCAB,), jnp.float32),
        pltpu.VMEM((HB,), jnp.int32),
        pltpu.VMEM((HB,), jnp.int32),
        pltpu.VMEM((HB,), jnp.float32),
    ],
    compiler_params=pltpu.CompilerParams(needs_layout_passes=False),
)
def _encode(xt_hbm, *rest):
    tables = rest[:N_FIELDS]
    out_hbm, row_v, idxa_v, idxb_v, och_v = rest[N_FIELDS:]
    wid = lax.axis_index("s") * NC + lax.axis_index("c")
    lo_u = (wid * NU) // NW
    hi_u = ((wid + 1) * NU) // NW
    UNROLL = 16

    for f in range(N_FIELDS):
        tbl = tables[f]
        dlo = jnp.clip(lo_u - f * DIM, 0, DIM)
        dhi = jnp.clip(hi_u - f * DIM, 0, DIM)

        @pl.when(dhi > dlo)
        def _(f=f):
            pltpu.sync_copy(xt_hbm.at[f, pl.ds(0, HB)], idxa_v)
            pltpu.sync_copy(xt_hbm.at[f, pl.ds(HB, HB)], idxb_v)

        def dbody(d, _, f=f, tbl=tbl):
            pltpu.sync_copy(tbl.at[d], row_v)
            for h, idx_v in ((0, idxa_v), (1, idxb_v)):

                def gbody(c, _, idx_v=idx_v):
                    base = c * (LANES * UNROLL)
                    for u in range(UNROLL):
                        iv = idx_v[pl.ds(base + u * LANES, LANES)]
                        och_v[pl.ds(base + u * LANES, LANES)] = (
                            plsc.load_gather(row_v, [iv]))
                    return 0

                lax.fori_loop(0, HB // (LANES * UNROLL), gbody, 0)
                pltpu.sync_copy(
                    och_v, out_hbm.at[f * DIM + d, pl.ds(h * HB, HB)])
            return 0

        lax.fori_loop(dlo, dhi, dbody, 0)


def kernel(x, table_0, table_1, table_2, table_3, table_4, table_5, table_6,
           table_7, table_8, table_9, table_10, table_11, table_12, table_13,
           table_14, table_15, table_16, table_17, table_18, table_19,
           table_20, table_21, table_22, table_23, table_24, table_25):
    tables = (table_0, table_1, table_2, table_3, table_4, table_5, table_6,
              table_7, table_8, table_9, table_10, table_11, table_12,
              table_13, table_14, table_15, table_16, table_17, table_18,
              table_19, table_20, table_21, table_22, table_23, table_24,
              table_25)
    out_t = _encode(x.T, *(t.T for t in tables))
    return out_t.T.reshape(BATCH, N_FIELDS * DIM)
